# baseline (speedup 1.0000x reference)
import functools

import jax
import jax.numpy as jnp
from jax import lax
from jax.experimental import pallas as pl
from jax.experimental.pallas import tpu as pltpu
from jax.experimental.pallas import tpu_sc as plsc


def kernel(x, token_table, pos_table):
    batch, seq = x.shape
    vocab, dim = token_table.shape
    n = batch * seq

    mesh = plsc.VectorSubcoreMesh(core_axis_name="c", subcore_axis_name="s")
    nw = mesh.num_cores * mesh.num_subcores
    n_per_w = n // nw
    assert n % nw == 0 and seq % n_per_w == 0 and dim % 16 == 0

    @functools.partial(
        pl.kernel,
        out_type=jax.ShapeDtypeStruct((n, dim), jnp.float32),
        mesh=mesh,
        compiler_params=pltpu.CompilerParams(use_tc_tiling_on_sc=False),
        scratch_types=[
            pltpu.VMEM((n_per_w,), jnp.int32),
            pltpu.VMEM((n_per_w, dim), jnp.float32),
            pltpu.VMEM((n_per_w, dim), jnp.float32),
            pltpu.SemaphoreType.DMA,
        ],
    )
    def _emb(idx_hbm, table_hbm, pos_hbm, out_hbm, idx_v, rows_v, pos_v, sem):
        wid = lax.axis_index("s") * mesh.num_cores + lax.axis_index("c")
        base = wid * n_per_w
        pos_base = lax.rem(base, seq)
        pltpu.sync_copy(idx_hbm.at[pl.ds(base, n_per_w)], idx_v)
        pos_cp = pltpu.async_copy(pos_hbm.at[pl.ds(pos_base, n_per_w)], pos_v, sem)
        gather = pltpu.async_copy(table_hbm.at[idx_v], rows_v, sem)
        pos_cp.wait()
        gather.wait()

        @pl.loop(0, n_per_w)
        def _add(r):
            for c in range(dim // 16):
                sl = (r, pl.ds(c * 16, 16))
                rows_v[sl] = rows_v[sl] + pos_v[sl]

        pltpu.sync_copy(rows_v, out_hbm.at[pl.ds(base, n_per_w)])

    out = _emb(x.reshape(n), token_table, pos_table)
    return out.reshape(batch, seq, dim)


# speedup vs baseline: 1.6925x; 1.6925x over previous
import functools

import jax
import jax.numpy as jnp
from jax import lax
from jax.experimental import pallas as pl
from jax.experimental.pallas import tpu as pltpu
from jax.experimental.pallas import tpu_sc as plsc


def kernel(x, token_table, pos_table):
    batch, seq = x.shape
    vocab, dim = token_table.shape
    n = batch * seq

    mesh = plsc.VectorSubcoreMesh(core_axis_name="c", subcore_axis_name="s")
    nw = mesh.num_cores * mesh.num_subcores
    n_per_w = n // nw
    assert n % nw == 0 and seq % n_per_w == 0 and dim % 16 == 0

    @functools.partial(
        pl.kernel,
        out_type=jax.ShapeDtypeStruct((n, dim), jnp.float32),
        mesh=mesh,
        compiler_params=pltpu.CompilerParams(use_tc_tiling_on_sc=True),
        scratch_types=[
            pltpu.VMEM((n_per_w,), jnp.int32),
            pltpu.VMEM((n_per_w, dim), jnp.float32),
            pltpu.VMEM((n_per_w, dim), jnp.float32),
            pltpu.SemaphoreType.DMA,
            pltpu.SemaphoreType.DMA,
        ],
    )
    def _emb(idx_hbm, table_hbm, pos_hbm, out_hbm, idx_v, rows_v, pos_v, sem, gsem):
        wid = lax.axis_index("s") * mesh.num_cores + lax.axis_index("c")
        base = wid * n_per_w
        pos_base = lax.rem(base, seq)
        pltpu.sync_copy(idx_hbm.at[pl.ds(base, n_per_w)], idx_v)
        pos_cp = pltpu.async_copy(pos_hbm.at[pl.ds(pos_base, n_per_w)], pos_v, sem)

        @pl.loop(0, n_per_w // 16)
        def _gather(g):
            vs = idx_v[pl.ds(g * 16, 16)]
            for l in range(16):
                pltpu.async_copy(
                    table_hbm.at[pl.ds(vs[l], 1)],
                    rows_v.at[pl.ds(g * 16 + l, 1)],
                    gsem,
                )

        pltpu.make_async_copy(
            table_hbm.at[pl.ds(0, n_per_w)], rows_v, gsem
        ).wait()
        pos_cp.wait()

        @pl.loop(0, n_per_w)
        def _add(r):
            for c in range(dim // 16):
                sl = (r, pl.ds(c * 16, 16))
                rows_v[sl] = rows_v[sl] + pos_v[sl]

        pltpu.sync_copy(rows_v, out_hbm.at[pl.ds(base, n_per_w)])

    out = _emb(x.reshape(n), token_table, pos_table)
    return out.reshape(batch, seq, dim)


# speedup vs baseline: 4.0833x; 2.4125x over previous
import functools

import jax
import jax.numpy as jnp
from jax import lax
from jax.experimental import pallas as pl
from jax.experimental.pallas import tpu as pltpu
from jax.experimental.pallas import tpu_sc as plsc


def kernel(x, token_table, pos_table):
    batch, seq = x.shape
    vocab, dim = token_table.shape
    n = batch * seq

    mesh = plsc.VectorSubcoreMesh(core_axis_name="c", subcore_axis_name="s")
    nw = mesh.num_cores * mesh.num_subcores
    n_per_w = n // nw
    assert n % nw == 0 and seq % n_per_w == 0 and dim % 16 == 0
    assert n_per_w % 16 == 0 and n_per_w % 128 == 0

    @functools.partial(
        pl.kernel,
        out_type=jax.ShapeDtypeStruct((n, dim), jnp.float32),
        mesh=mesh,
        compiler_params=pltpu.CompilerParams(
            use_tc_tiling_on_sc=True, needs_layout_passes=False
        ),
        scratch_types=[
            pltpu.VMEM((n_per_w,), jnp.int32),
            pltpu.VMEM((8, dim, 128), jnp.float32),
            pltpu.VMEM((n_per_w, dim), jnp.float32),
            pltpu.VMEM((dim, n_per_w), jnp.float32),
            pltpu.SemaphoreType.DMA,
            pltpu.SemaphoreType.DMA,
            pltpu.SemaphoreType.DMA,
        ],
    )
    def _emb(idx_hbm, tbl_hbm, pos_hbm, out_hbm, idx_v, blk_v, rows_v, pos_v,
             psem, sema, semb):
        wid = lax.axis_index("s") * mesh.num_cores + lax.axis_index("c")
        base = wid * n_per_w
        pos_base = lax.rem(base, seq)
        pltpu.sync_copy(idx_hbm.at[pl.ds(base, n_per_w)], idx_v)
        pos_cp = pltpu.async_copy(
            pos_hbm.at[:, pl.ds(pos_base, n_per_w)], pos_v, psem
        )
        pos_cp.wait()

        dvecs = [lax.iota(jnp.int32, 16) + 16 * c for c in range(dim // 16)]

        def fire(vs, l, slot, sem):
            vb = pl.multiple_of((vs[l] >> 7) * 128, 128)
            pltpu.async_copy(
                tbl_hbm.at[:, pl.ds(vb, 128)], blk_v.at[slot], sem
            )

        def wait_quad(sem):
            for _ in range(4):
                pltpu.make_async_copy(
                    tbl_hbm.at[:, pl.ds(0, 128)], blk_v.at[0], sem
                ).wait()

        def extract(g, vs, l, slot):
            i = g * 16 + l
            p = jnp.full((16,), vs[l] & 127, jnp.int32)
            ivec = jnp.full((16,), i, jnp.int32)
            for c in range(dim // 16):
                tok = plsc.load_gather(blk_v.at[slot], [dvecs[c], p])
                pos = plsc.load_gather(pos_v, [dvecs[c], ivec])
                rows_v[i, pl.ds(16 * c, 16)] = tok + pos

        @pl.loop(0, n_per_w // 16)
        def _block(g):
            vs = idx_v[pl.ds(g * 16, 16)]
            for l in range(4):
                fire(vs, l, l, sema)
            for l in range(4, 8):
                fire(vs, l, l, semb)
            wait_quad(sema)
            for l in range(4):
                extract(g, vs, l, l)
            for l in range(8, 12):
                fire(vs, l, l - 8, sema)
            wait_quad(semb)
            for l in range(4, 8):
                extract(g, vs, l, l)
            for l in range(12, 16):
                fire(vs, l, l - 8, semb)
            wait_quad(sema)
            for l in range(8, 12):
                extract(g, vs, l, l - 8)
            wait_quad(semb)
            for l in range(12, 16):
                extract(g, vs, l, l - 8)

        pltpu.sync_copy(rows_v, out_hbm.at[pl.ds(base, n_per_w)])

    out = _emb(x.reshape(n), token_table.T, pos_table.T)
    return out.reshape(batch, seq, dim)


# speedup vs baseline: 4.1385x; 1.0135x over previous
import functools

import jax
import jax.numpy as jnp
from jax import lax
from jax.experimental import pallas as pl
from jax.experimental.pallas import tpu as pltpu
from jax.experimental.pallas import tpu_sc as plsc


def kernel(x, token_table, pos_table):
    batch, seq = x.shape
    vocab, dim = token_table.shape
    n = batch * seq

    mesh = plsc.VectorSubcoreMesh(core_axis_name="c", subcore_axis_name="s")
    nw = mesh.num_cores * mesh.num_subcores
    n_per_w = n // nw
    assert n % nw == 0 and seq % n_per_w == 0 and dim % 16 == 0
    assert n_per_w % 16 == 0 and n_per_w % 8 == 0

    @functools.partial(
        pl.kernel,
        out_type=jax.ShapeDtypeStruct((n, dim), jnp.float32),
        mesh=mesh,
        compiler_params=pltpu.CompilerParams(
            use_tc_tiling_on_sc=True, needs_layout_passes=False
        ),
        scratch_types=[
            pltpu.VMEM((n_per_w,), jnp.int32),
            pltpu.VMEM((8, dim, 128), jnp.float32),
            pltpu.VMEM((n_per_w, dim), jnp.float32),
            pltpu.SemaphoreType.DMA,
            pltpu.SemaphoreType.DMA,
            pltpu.SemaphoreType.DMA,
        ],
    )
    def _emb(idx_hbm, tbl_hbm, pos_hbm, out_hbm, idx_v, blk_v, rows_v,
             psem, sa, sb):
        wid = lax.axis_index("s") * mesh.num_cores + lax.axis_index("c")
        base = wid * n_per_w
        pos_base = lax.rem(base, seq)
        pltpu.sync_copy(idx_hbm.at[pl.ds(base, n_per_w)], idx_v)
        pos_cp = pltpu.async_copy(
            pos_hbm.at[pl.ds(pos_base, n_per_w)], rows_v, psem
        )

        sems = [sa, sb]
        dvecs = [lax.iota(jnp.int32, 16) + 16 * c for c in range(dim // 16)]
        nq = n_per_w // 4
        vs = [idx_v[pl.ds(g * 16, 16)] for g in range(n_per_w // 16)]

        def fire(q):
            slab = q % 2
            for j in range(4):
                i = 4 * q + j
                v = vs[i // 16][i % 16]
                vb = pl.multiple_of((v >> 7) * 128, 128)
                pltpu.async_copy(
                    tbl_hbm.at[:, pl.ds(vb, 128)],
                    blk_v.at[4 * slab + j],
                    sems[slab],
                )

        def wait_quad(q):
            for _ in range(4):
                pltpu.make_async_copy(
                    tbl_hbm.at[:, pl.ds(0, 128)], blk_v.at[0], sems[q % 2]
                ).wait()

        def extract(q):
            slab = q % 2
            for j in range(4):
                i = 4 * q + j
                v = vs[i // 16][i % 16]
                p = jnp.full((16,), v & 127, jnp.int32)
                for c in range(dim // 16):
                    tok = plsc.load_gather(blk_v.at[4 * slab + j], [dvecs[c], p])
                    sl = (i, pl.ds(16 * c, 16))
                    rows_v[sl] = rows_v[sl] + tok

        fire(0)
        fire(1)
        pos_cp.wait()
        for q in range(nq):
            wait_quad(q)
            extract(q)
            if q + 2 < nq:
                fire(q + 2)

        pltpu.sync_copy(rows_v, out_hbm.at[pl.ds(base, n_per_w)])

    out = _emb(x.reshape(n), token_table.T, pos_table)
    return out.reshape(batch, seq, dim)
